# baseline (speedup 1.0000x reference)
import functools

import jax
import jax.numpy as jnp
from jax import lax
from jax.experimental import pallas as pl
from jax.experimental.pallas import tpu as pltpu
from jax.experimental.pallas import tpu_sc as plsc

B, L = 4096, 200
V, FEA, K = 1000000, 64, 128

NC, NS = 2, 16
NW = NC * NS
BPW = B // NW
BPC = B // NC
TRASH = 2 * BPC
NCH = 13
COLS = tuple(min(16 * j, L - 16) for j in range(NCH))


def _sc_pool(log, mask, emb_pairs):
    mesh = plsc.VectorSubcoreMesh(core_axis_name="c", subcore_axis_name="s")

    @functools.partial(
        pl.kernel,
        out_type=jax.ShapeDtypeStruct((2 * B, 2 * FEA), jnp.float32),
        mesh=mesh,
        scratch_types=[
            pltpu.VMEM((BPW, L), jnp.int32),
            pltpu.VMEM((BPW,), jnp.int32),
            pltpu.VMEM((BPW, 2 * FEA), jnp.float32),
            pltpu.VMEM_SHARED((2 * BPC + 8, 2 * FEA), jnp.float32),
        ]
        + [pltpu.VMEM((16, 2 * FEA), jnp.float32) for _ in range(NCH)]
        + [pltpu.SemaphoreType.DMA for _ in range(2 * NCH)],
    )
    def k(table_h, log_h, mask_h, out_h, log_v, mask_v, zbuf, acc, *rest):
        bufs = rest[:NCH]
        sem_g = rest[NCH:2 * NCH]
        sem_s = rest[2 * NCH:]
        c = lax.axis_index("c")
        s = lax.axis_index("s")
        gbase = c * BPC + s * BPW
        dbase = 2 * s * BPW

        pltpu.sync_copy(log_h.at[pl.ds(gbase, BPW), :], log_v)
        pltpu.sync_copy(mask_h.at[pl.ds(gbase, BPW)], mask_v)

        def zrow(r, carry):
            for j in range(2 * FEA // 16):
                zbuf[r, pl.ds(j * 16, 16)] = jnp.zeros((16,), jnp.float32)
            return carry
        lax.fori_loop(0, BPW, zrow, 0)
        pltpu.sync_copy(zbuf, acc.at[pl.ds(dbase, BPW), :])
        pltpu.sync_copy(zbuf, acc.at[pl.ds(dbase + BPW, BPW), :])

        iota = lax.iota(jnp.int32, 16)
        trash_vec = jnp.full((16,), TRASH, jnp.int32)

        def gwait(j):
            pltpu.make_async_copy(table_h.at[iota], bufs[j], sem_g[j]).wait()

        def swait(j):
            pltpu.make_async_copy(bufs[j], acc.at[iota], sem_s[j]).wait()

        def rowbody(r, carry):
            for j in range(NCH):
                @pl.when(r > 0)
                def _():
                    swait(j)
                ivec = log_v[r, pl.ds(COLS[j], 16)]
                pltpu.async_copy(table_h.at[ivec >> 1], bufs[j], sem_g[j])
            mvec = mask_v[pl.ds((r >> 4) << 4, 16)]
            lens = mvec.at[jnp.full((16,), r & 15, jnp.int32)].get(
                mode="promise_in_bounds")
            drow = jnp.full((16,), dbase + 2 * r, jnp.int32)
            for j in range(NCH):
                gwait(j)
                ivec = log_v[r, pl.ds(COLS[j], 16)]
                valid = (iota + COLS[j]) < lens
                if COLS[j] != 16 * j:
                    valid = valid & (iota >= (16 * j - COLS[j]))
                dval = jnp.where(valid, drow + (ivec & 1), trash_vec)
                pltpu.async_copy(bufs[j], acc.at[dval], sem_s[j], add=True)
            return carry
        lax.fori_loop(0, BPW, rowbody, 0)
        for j in range(NCH):
            swait(j)

        pltpu.sync_copy(acc.at[pl.ds(dbase, 2 * BPW), :],
                        out_h.at[pl.ds(2 * gbase, 2 * BPW), :])

    return k(emb_pairs, log, mask)


def _tc_project(pooled2, mask, W, b):
    BLK = 256

    def body(p_ref, m_ref, w_ref, b_ref, o_ref):
        even = p_ref[:, 0, :FEA]
        odd = p_ref[:, 1, FEA:]
        x = even + odd
        lens = m_ref[...].astype(jnp.float32)
        x = x / lens
        o_ref[...] = (
            jnp.dot(x, w_ref[...], preferred_element_type=jnp.float32) + b_ref[...]
        )

    return pl.pallas_call(
        body,
        grid=(B // BLK,),
        in_specs=[
            pl.BlockSpec((BLK, 2, 2 * FEA), lambda i: (i, 0, 0)),
            pl.BlockSpec((BLK, 1), lambda i: (i, 0)),
            pl.BlockSpec((FEA, K), lambda i: (0, 0)),
            pl.BlockSpec((1, K), lambda i: (0, 0)),
        ],
        out_specs=pl.BlockSpec((BLK, K), lambda i: (i, 0)),
        out_shape=jax.ShapeDtypeStruct((B, K), jnp.float32),
    )(pooled2, mask.reshape(B, 1), W, b.reshape(1, K))


def kernel(log, mask, emb_table, W, b):
    log = log.astype(jnp.int32)
    mask = mask.astype(jnp.int32)
    emb_pairs = emb_table.reshape(V // 2, 2 * FEA)
    pooled = _sc_pool(log, mask, emb_pairs)
    pooled2 = pooled.reshape(B, 2, 2 * FEA)
    return _tc_project(pooled2, mask, W, b)


# speedup vs baseline: 1.5278x; 1.5278x over previous
import functools

import jax
import jax.numpy as jnp
from jax import lax
from jax.experimental import pallas as pl
from jax.experimental.pallas import tpu as pltpu
from jax.experimental.pallas import tpu_sc as plsc

B, L = 4096, 200
V, FEA, K = 1000000, 64, 128
H = 1 << 19

NC, NS = 2, 16
NW = NC * NS
BPW = B // NW
BPC = B // NC
TRASH = 2 * BPC
NCH = 13
COLS = tuple(min(16 * j, L - 16) for j in range(NCH))


def _sc_pool(log, mask, emb_pairs):
    mesh = plsc.VectorSubcoreMesh(core_axis_name="c", subcore_axis_name="s")

    @functools.partial(
        pl.kernel,
        out_type=jax.ShapeDtypeStruct((2 * B, 2 * FEA), jnp.float32),
        mesh=mesh,
        scratch_types=[
            pltpu.VMEM((BPW, L), jnp.int32),
            pltpu.VMEM((BPW,), jnp.int32),
            pltpu.VMEM((BPW, 2 * FEA), jnp.float32),
            pltpu.VMEM_SHARED((2 * BPC + 8, 2 * FEA), jnp.float32),
        ]
        + [pltpu.VMEM((16, 2 * FEA), jnp.float32) for _ in range(NCH)]
        + [pltpu.SemaphoreType.DMA for _ in range(2 * NCH)],
    )
    def k(table_h, log_h, mask_h, out_h, log_v, mask_v, zbuf, acc, *rest):
        bufs = rest[:NCH]
        sem_g = rest[NCH:2 * NCH]
        sem_s = rest[2 * NCH:]
        c = lax.axis_index("c")
        s = lax.axis_index("s")
        gbase = c * BPC + s * BPW
        dbase = 2 * s * BPW

        pltpu.sync_copy(log_h.at[pl.ds(gbase, BPW), :], log_v)
        pltpu.sync_copy(mask_h.at[pl.ds(gbase, BPW)], mask_v)

        def zrow(r, carry):
            for j in range(2 * FEA // 16):
                zbuf[r, pl.ds(j * 16, 16)] = jnp.zeros((16,), jnp.float32)
            return carry
        lax.fori_loop(0, BPW, zrow, 0)
        pltpu.sync_copy(zbuf, acc.at[pl.ds(dbase, BPW), :])
        pltpu.sync_copy(zbuf, acc.at[pl.ds(dbase + BPW, BPW), :])

        iota = lax.iota(jnp.int32, 16)
        trash_vec = jnp.full((16,), TRASH, jnp.int32)

        def gwait(j):
            pltpu.make_async_copy(table_h.at[iota], bufs[j], sem_g[j]).wait()

        def swait(j):
            pltpu.make_async_copy(bufs[j], acc.at[iota], sem_s[j]).wait()

        def rowbody(r, carry):
            for j in range(NCH):
                @pl.when(r > 0)
                def _():
                    swait(j)
                ivec = log_v[r, pl.ds(COLS[j], 16)]
                pltpu.async_copy(table_h.at[ivec & (H - 1)], bufs[j], sem_g[j])
            mvec = mask_v[pl.ds((r >> 4) << 4, 16)]
            lens = mvec.at[jnp.full((16,), r & 15, jnp.int32)].get(
                mode="promise_in_bounds")
            drow = jnp.full((16,), dbase + 2 * r, jnp.int32)
            for j in range(NCH):
                gwait(j)
                ivec = log_v[r, pl.ds(COLS[j], 16)]
                valid = (iota + COLS[j]) < lens
                if COLS[j] != 16 * j:
                    valid = valid & (iota >= (16 * j - COLS[j]))
                dval = jnp.where(valid, drow + (ivec >> 19), trash_vec)
                pltpu.async_copy(bufs[j], acc.at[dval], sem_s[j], add=True)
            return carry
        lax.fori_loop(0, BPW, rowbody, 0)
        for j in range(NCH):
            swait(j)

        pltpu.sync_copy(acc.at[pl.ds(dbase, 2 * BPW), :],
                        out_h.at[pl.ds(2 * gbase, 2 * BPW), :])

    return k(emb_pairs, log, mask)


def _tc_pairs(tt):
    CB = 4096
    G = H // CB

    def body(lo_ref, hi_ref, o_ref):
        ylo = jnp.transpose(lo_ref[...])
        yhi = jnp.transpose(hi_ref[...])
        o_ref[...] = jnp.concatenate([ylo, yhi], axis=1)

    return pl.pallas_call(
        body,
        grid=(G,),
        in_specs=[
            pl.BlockSpec((FEA, CB), lambda j: (0, j)),
            pl.BlockSpec((FEA, CB), lambda j: (0, jnp.minimum(j + G, V // CB))),
        ],
        out_specs=pl.BlockSpec((CB, 2 * FEA), lambda j: (j, 0)),
        out_shape=jax.ShapeDtypeStruct((H, 2 * FEA), jnp.float32),
    )(tt, tt)


def _tc_project(pooled2, mask, W, b):
    BLK = 256

    def body(p_ref, m_ref, w_ref, b_ref, o_ref):
        even = p_ref[:, 0, :FEA]
        odd = p_ref[:, 1, FEA:]
        x = even + odd
        lens = m_ref[...].astype(jnp.float32)
        x = x / lens
        o_ref[...] = (
            jnp.dot(x, w_ref[...], preferred_element_type=jnp.float32) + b_ref[...]
        )

    return pl.pallas_call(
        body,
        grid=(B // BLK,),
        in_specs=[
            pl.BlockSpec((BLK, 2, 2 * FEA), lambda i: (i, 0, 0)),
            pl.BlockSpec((BLK, 1), lambda i: (i, 0)),
            pl.BlockSpec((FEA, K), lambda i: (0, 0)),
            pl.BlockSpec((1, K), lambda i: (0, 0)),
        ],
        out_specs=pl.BlockSpec((BLK, K), lambda i: (i, 0)),
        out_shape=jax.ShapeDtypeStruct((B, K), jnp.float32),
    )(pooled2, mask.reshape(B, 1), W, b.reshape(1, K))


def kernel(log, mask, emb_table, W, b):
    log = log.astype(jnp.int32)
    mask = mask.astype(jnp.int32)
    emb_pairs = _tc_pairs(jnp.transpose(emb_table))
    pooled = _sc_pool(log, mask, emb_pairs)
    pooled2 = pooled.reshape(B, 2, 2 * FEA)
    return _tc_project(pooled2, mask, W, b)


# speedup vs baseline: 1.7831x; 1.1671x over previous
import functools

import jax
import jax.numpy as jnp
from jax import lax
from jax.experimental import pallas as pl
from jax.experimental.pallas import tpu as pltpu
from jax.experimental.pallas import tpu_sc as plsc

B, L = 4096, 200
V, FEA, K = 1000000, 64, 128
H = 1 << 19

NC, NS = 2, 16
NW = NC * NS
BPW = B // NW
BPC = B // NC
TRASH = 2 * BPC
NCH = 13
COLS = tuple(min(16 * j, L - 16) for j in range(NCH))


def _sc_pool(log, mask, emb_pairs):
    mesh = plsc.VectorSubcoreMesh(core_axis_name="c", subcore_axis_name="s")

    @functools.partial(
        pl.kernel,
        out_type=jax.ShapeDtypeStruct((2 * B, 2 * FEA), jnp.float32),
        mesh=mesh,
        scratch_types=[
            pltpu.VMEM((BPW, L), jnp.int32),
            pltpu.VMEM((BPW,), jnp.int32),
            pltpu.VMEM_SHARED((2 * BPC + NS, 2 * FEA), jnp.float32),
        ]
        + [pltpu.VMEM((16, 2 * FEA), jnp.float32) for _ in range(2 * NCH)]
        + [pltpu.SemaphoreType.DMA for _ in range(4)],
    )
    def k(table_h, log_h, mask_h, out_h, log_v, mask_v, acc, *rest):
        bufs = rest[:2 * NCH]
        sem_g = rest[2 * NCH:2 * NCH + 2]
        sem_s = rest[2 * NCH + 2:]
        c = lax.axis_index("c")
        s = lax.axis_index("s")
        gbase = c * BPC + s * BPW
        dbase = 2 * s * BPW

        pltpu.sync_copy(log_h.at[pl.ds(gbase, BPW), :], log_v)
        pltpu.sync_copy(mask_h.at[pl.ds(gbase, BPW)], mask_v)

        def zrow(r, carry):
            for j in range(2 * FEA // 16):
                bufs[0][r, pl.ds(j * 16, 16)] = jnp.zeros((16,), jnp.float32)
            return carry
        lax.fori_loop(0, 16, zrow, 0)

        def zcopy(kk, carry):
            pltpu.sync_copy(bufs[0], acc.at[pl.ds(dbase + 16 * kk, 16), :])
            return carry
        lax.fori_loop(0, 2 * BPW // 16, zcopy, 0)

        iota = lax.iota(jnp.int32, 16)
        trash_vec = jnp.full((16,), TRASH, jnp.int32) + s

        def issue_g(r, ss):
            for j in range(NCH):
                ivec = log_v[r, pl.ds(COLS[j], 16)]
                pltpu.async_copy(table_h.at[ivec & (H - 1)],
                                 bufs[NCH * ss + j], sem_g[ss])

        def drain_g(ss):
            for j in range(NCH):
                pltpu.make_async_copy(table_h.at[iota],
                                      bufs[NCH * ss + j], sem_g[ss]).wait()

        def issue_s(r, ss):
            mvec = mask_v[pl.ds((r >> 4) << 4, 16)]
            lens = mvec.at[jnp.full((16,), r & 15, jnp.int32)].get(
                mode="promise_in_bounds")
            drow = jnp.full((16,), dbase + 2 * r, jnp.int32)
            for j in range(NCH):
                ivec = log_v[r, pl.ds(COLS[j], 16)]
                valid = (iota + COLS[j]) < lens
                if COLS[j] != 16 * j:
                    valid = valid & (iota >= (16 * j - COLS[j]))
                dval = jnp.where(valid, drow + (ivec >> 19), trash_vec)
                pltpu.async_copy(bufs[NCH * ss + j], acc.at[dval],
                                 sem_s[ss], add=True)

        def drain_s(ss):
            for j in range(NCH):
                pltpu.make_async_copy(bufs[NCH * ss + j],
                                      acc.at[iota], sem_s[ss]).wait()

        issue_g(0, 0)

        def blk(rb2, carry):
            rb = rb2 * 2
            for p in (0, 1):
                r = rb + p

                @pl.when(r > 0)
                def _():
                    drain_s(1 - p)

                @pl.when(r + 1 < BPW)
                def _():
                    issue_g(r + 1, 1 - p)

                drain_g(p)
                issue_s(r, p)
            return carry
        lax.fori_loop(0, BPW // 2, blk, 0)
        drain_s((BPW - 1) & 1)

        pltpu.sync_copy(acc.at[pl.ds(dbase, 2 * BPW), :],
                        out_h.at[pl.ds(2 * gbase, 2 * BPW), :])

    return k(emb_pairs, log, mask)


def _tc_pairs(tt):
    CB = 4096
    G = H // CB

    def body(lo_ref, hi_ref, o_ref):
        ylo = jnp.transpose(lo_ref[...])
        yhi = jnp.transpose(hi_ref[...])
        o_ref[...] = jnp.concatenate([ylo, yhi], axis=1)

    return pl.pallas_call(
        body,
        grid=(G,),
        in_specs=[
            pl.BlockSpec((FEA, CB), lambda j: (0, j)),
            pl.BlockSpec((FEA, CB), lambda j: (0, jnp.minimum(j + G, V // CB))),
        ],
        out_specs=pl.BlockSpec((CB, 2 * FEA), lambda j: (j, 0)),
        out_shape=jax.ShapeDtypeStruct((H, 2 * FEA), jnp.float32),
    )(tt, tt)


def _tc_project(pooled2, mask, W, b):
    BLK = 256

    def body(p_ref, m_ref, w_ref, b_ref, o_ref):
        even = p_ref[:, 0, :FEA]
        odd = p_ref[:, 1, FEA:]
        x = even + odd
        lens = m_ref[...].astype(jnp.float32)
        x = x / lens
        o_ref[...] = (
            jnp.dot(x, w_ref[...], preferred_element_type=jnp.float32) + b_ref[...]
        )

    return pl.pallas_call(
        body,
        grid=(B // BLK,),
        in_specs=[
            pl.BlockSpec((BLK, 2, 2 * FEA), lambda i: (i, 0, 0)),
            pl.BlockSpec((BLK, 1), lambda i: (i, 0)),
            pl.BlockSpec((FEA, K), lambda i: (0, 0)),
            pl.BlockSpec((1, K), lambda i: (0, 0)),
        ],
        out_specs=pl.BlockSpec((BLK, K), lambda i: (i, 0)),
        out_shape=jax.ShapeDtypeStruct((B, K), jnp.float32),
    )(pooled2, mask.reshape(B, 1), W, b.reshape(1, K))


def kernel(log, mask, emb_table, W, b):
    log = log.astype(jnp.int32)
    mask = mask.astype(jnp.int32)
    emb_pairs = _tc_pairs(jnp.transpose(emb_table))
    pooled = _sc_pool(log, mask, emb_pairs)
    pooled2 = pooled.reshape(B, 2, 2 * FEA)
    return _tc_project(pooled2, mask, W, b)
